# Initial kernel scaffold; baseline (speedup 1.0000x reference)
import jax
import jax.numpy as jnp
from jax.experimental import pallas as pl


def kernel(w1, b1, w2t, b2v, fc1_w, fc1_b, fc2_w, fc2_b, x_nchw):
    raise NotImplementedError("write your pallas kernel here")


# speedup vs baseline: 1.9659x; 1.9659x over previous
import numpy as np

import jax
import jax.numpy as jnp
from jax.experimental import pallas as pl
from jax.experimental.pallas import tpu as pltpu


BB = 256


def _conv1_band_index():
    idx = np.full((480, 168), 250, np.int32)
    for u in range(2):
        for v in range(2):
            for c in range(10):
                for jp in range(12):
                    m = u * 240 + v * 120 + c * 12 + jp
                    for di in range(5):
                        for dj in range(5):
                            idx[m, (u + di) * 28 + 2 * jp + v + dj] = (
                                c * 25 + di * 5 + dj)
    return idx


_A2_IDX = _conv1_band_index()
_C2_COLS = np.array([ci * 16 + w for ci in range(10) for w in range(12)],
                    np.int32)


def _fused_kernel(xp_ref, a2_ref, b1v_ref, w2c_ref, b2v_ref,
                  fc1w_ref, fc1b_ref, fc2w_ref, fc2b_ref, o_ref, p1_ref):

    for p in range(12):
        win = xp_ref[p * 56:p * 56 + 168, :]
        r = jnp.dot(a2_ref[...], win,
                    preferred_element_type=jnp.float32)
        m = jnp.maximum(jnp.maximum(r[0:120], r[120:240]),
                        jnp.maximum(r[240:360], r[360:480]))
        p1_ref[p * 120:(p + 1) * 120, :] = jnp.maximum(m + b1v_ref[...], 0.0)

    rmax = []
    for i in range(8):
        c2 = jnp.dot(w2c_ref[...], p1_ref[i * 120:i * 120 + 600, :],
                     preferred_element_type=jnp.float32)
        rmax.append(jnp.maximum(c2[0:80], c2[80:160]))

    b2v = b2v_ref[...]
    flat = jnp.concatenate(
        [jnp.maximum(jnp.maximum(rmax[2 * ip], rmax[2 * ip + 1]) + b2v, 0.0)
         for ip in range(4)], axis=0)

    h1 = jnp.maximum(
        jnp.dot(fc1w_ref[...], flat, preferred_element_type=jnp.float32)
        + fc1b_ref[...], 0.0)
    z = (jnp.dot(fc2w_ref[...], h1, preferred_element_type=jnp.float32)
         + fc2b_ref[...])

    zmax = jnp.max(z, axis=0, keepdims=True)
    s = z - zmax
    lse = jnp.log(jnp.sum(jnp.exp(s), axis=0, keepdims=True))
    o_ref[...] = s - lse


def kernel(w1, b1, w2t, b2v, fc1_w, fc1_b, fc2_w, fc2_b, x_nchw):
    n = x_nchw.shape[0]
    npad = ((n + BB - 1) // BB) * BB

    x2 = x_nchw[:, 0].reshape(n, 784)
    if npad != n:
        x2 = jnp.concatenate(
            [x2, jnp.zeros((npad - n, 784), x2.dtype)], axis=0)
    xp = x2.T

    w1x = jnp.concatenate([w1, jnp.zeros((1,), jnp.float32)])
    a2 = w1x[_A2_IDX]
    b1v = jnp.repeat(b1, 12).reshape(120, 1)
    w2c = jnp.transpose(w2t[:, :, _C2_COLS], (1, 0, 2)).reshape(160, 600)

    out = pl.pallas_call(
        _fused_kernel,
        out_shape=jax.ShapeDtypeStruct((10, npad), jnp.float32),
        grid=(npad // BB,),
        in_specs=[
            pl.BlockSpec((784, BB), lambda i: (0, i)),
            pl.BlockSpec((480, 168), lambda i: (0, 0)),
            pl.BlockSpec((120, 1), lambda i: (0, 0)),
            pl.BlockSpec((160, 600), lambda i: (0, 0)),
            pl.BlockSpec((80, 1), lambda i: (0, 0)),
            pl.BlockSpec((50, 320), lambda i: (0, 0)),
            pl.BlockSpec((50, 1), lambda i: (0, 0)),
            pl.BlockSpec((10, 50), lambda i: (0, 0)),
            pl.BlockSpec((10, 1), lambda i: (0, 0)),
        ],
        out_specs=pl.BlockSpec((10, BB), lambda i: (0, i)),
        scratch_shapes=[pltpu.VMEM((1440, BB), jnp.float32)],
        compiler_params=pltpu.CompilerParams(
            dimension_semantics=("parallel",),
            vmem_limit_bytes=32 * 1024 * 1024),
    )(xp, a2, b1v, w2c, b2v, fc1_w, fc1_b, fc2_w, fc2_b)

    return out[:, :n].T
